# baseline (speedup 1.0000x reference)
import jax
import jax.numpy as jnp
from jax.experimental import pallas as pl

_B, _D, _H, _A = 16384, 500, 500, 8
_BT = 1024


def _fused_body(x_ref, u_ref, w1_ref, b1_ref, w2_ref, b2_ref, out_ref):
    x = x_ref[...]
    h = jnp.dot(x, w1_ref[...], preferred_element_type=jnp.float32)
    h = jnp.maximum(h + b1_ref[...], 0.0)
    z = jnp.dot(h, w2_ref[...], preferred_element_type=jnp.float32)
    z = z + b2_ref[...]
    m = jnp.max(z, axis=1, keepdims=True)
    e = jnp.exp(z - m)
    prob = e / jnp.sum(e, axis=1, keepdims=True)
    logits = jnp.log(prob + 1e-20)
    m2 = jnp.max(logits, axis=1, keepdims=True)
    pdf = jnp.exp(logits - m2)
    tri = (jax.lax.broadcasted_iota(jnp.int32, (_A, _A), 0)
           <= jax.lax.broadcasted_iota(jnp.int32, (_A, _A), 1)
           ).astype(jnp.float32)
    cdf = jnp.dot(pdf, tri, preferred_element_type=jnp.float32,
                  precision=jax.lax.Precision.HIGHEST)
    total = jnp.max(cdf, axis=1, keepdims=True)
    us = u_ref[...] * total
    cnt = jnp.sum((cdf <= us).astype(jnp.int32), axis=1)
    out_ref[...] = cnt[:, None]


def kernel(inputs, u, W1, b1, W2, b2):
    b1r = b1.reshape(1, _H)
    b2r = b2.reshape(1, _A)
    grid = (_B // _BT,)
    out = pl.pallas_call(
        _fused_body,
        grid=grid,
        in_specs=[
            pl.BlockSpec((_BT, _D), lambda i: (i, 0)),
            pl.BlockSpec((_BT, 1), lambda i: (i, 0)),
            pl.BlockSpec((_D, _H), lambda i: (0, 0)),
            pl.BlockSpec((1, _H), lambda i: (0, 0)),
            pl.BlockSpec((_D, _A), lambda i: (0, 0)),
            pl.BlockSpec((1, _A), lambda i: (0, 0)),
        ],
        out_specs=pl.BlockSpec((_BT, 1), lambda i: (i, 0)),
        out_shape=jax.ShapeDtypeStruct((_B, 1), jnp.int32),
    )(inputs, u, W1, b1r, W2, b2r)
    return out.reshape(_B).astype(jnp.int64)


# speedup vs baseline: 4.2695x; 4.2695x over previous
import jax
import jax.numpy as jnp
from jax.experimental import pallas as pl

_B, _D, _H, _A = 16384, 500, 500, 8
_BT = 1024


def _fused_body(xt_ref, ut_ref, w1_ref, b1_ref, w2t_ref, b2_ref, out_ref):
    xt = xt_ref[...]
    ht = jax.lax.dot_general(
        w1_ref[...], xt, (((0,), (0,)), ((), ())),
        preferred_element_type=jnp.float32)
    ht = jnp.maximum(ht + b1_ref[...], 0.0)
    zt = jnp.dot(w2t_ref[...], ht,
                 preferred_element_type=jnp.float32)
    zt = zt + b2_ref[...]
    m = jnp.max(zt, axis=0, keepdims=True)
    e = jnp.exp(zt - m)
    prob = e / jnp.sum(e, axis=0, keepdims=True)
    logits = jnp.log(prob + 1e-20)
    m2 = jnp.max(logits, axis=0, keepdims=True)
    pdf = jnp.exp(logits - m2)
    tri = (jax.lax.broadcasted_iota(jnp.int32, (_A, _A), 1)
           <= jax.lax.broadcasted_iota(jnp.int32, (_A, _A), 0)
           ).astype(jnp.float32)
    cdf = jnp.dot(tri, pdf, preferred_element_type=jnp.float32,
                  precision=jax.lax.Precision.HIGHEST)
    total = jnp.max(cdf, axis=0, keepdims=True)
    us = ut_ref[...] * total
    cnt = jnp.sum((cdf <= us).astype(jnp.int32), axis=0, keepdims=True)
    out_ref[...] = cnt


def kernel(inputs, u, W1, b1, W2, b2):
    xt = inputs.T
    ut = u.T
    w2t = W2.T
    b1r = b1.reshape(_H, 1)
    b2r = b2.reshape(_A, 1)
    grid = (_B // _BT,)
    out = pl.pallas_call(
        _fused_body,
        grid=grid,
        in_specs=[
            pl.BlockSpec((_D, _BT), lambda i: (0, i)),
            pl.BlockSpec((1, _BT), lambda i: (0, i)),
            pl.BlockSpec((_D, _H), lambda i: (0, 0)),
            pl.BlockSpec((_H, 1), lambda i: (0, 0)),
            pl.BlockSpec((_A, _D), lambda i: (0, 0)),
            pl.BlockSpec((_A, 1), lambda i: (0, 0)),
        ],
        out_specs=pl.BlockSpec((1, _BT), lambda i: (0, i)),
        out_shape=jax.ShapeDtypeStruct((1, _B), jnp.int32),
    )(xt, ut, W1, b1r, w2t, b2r)
    return out.reshape(_B).astype(jnp.int64)
